# Initial kernel scaffold; baseline (speedup 1.0000x reference)
import jax
import jax.numpy as jnp
from jax.experimental import pallas as pl


def kernel(word_ids, context_ids, context_masks, neg_ids, emb_w, global_w, disamb_w):
    raise NotImplementedError("write your pallas kernel here")


# speedup vs baseline: 1.8642x; 1.8642x over previous
import functools

import jax
import jax.numpy as jnp
import numpy as np
from jax import lax
from jax.experimental import pallas as pl
from jax.experimental.pallas import tpu as pltpu
from jax.experimental.pallas import tpu_sc as plsc

NUM_SENSE = 3
EMB_DIM = 64

NC = 2
NS = 16
NW = NC * NS
CHUNK = 128

_TINY = float(np.finfo(np.float32).tiny)


def _gather_body(emb_w, dis_w, glob, ctx_s0, ctx_s1, ctx_s2, ctx_g,
                 neg_g, word_s0, word_s1, word_s2,
                 cs0, cs1, cs2, cd0, cd1, cd2, cg, ng,
                 ws0, ws1, ws2, wd0, wd1, wd2,
                 i_cs0, i_cs1, i_cs2, i_cg, i_ng, i_w0, i_w1, i_w2,
                 b_e0, b_e1, b_e2, b_d0, b_d1, b_d2, b_g,
                 s_e0, s_e1, s_e2, s_d0, s_d1, s_d2, s_g):
    wid = lax.axis_index("s") * NC + lax.axis_index("c")
    n_ctx = ctx_g.shape[1]
    n_neg = neg_g.shape[1]

    pltpu.sync_copy(ctx_s0.at[wid], i_cs0)
    pltpu.sync_copy(ctx_s1.at[wid], i_cs1)
    pltpu.sync_copy(ctx_s2.at[wid], i_cs2)
    pltpu.sync_copy(ctx_g.at[wid], i_cg)
    pltpu.sync_copy(neg_g.at[wid], i_ng)
    pltpu.sync_copy(word_s0.at[wid], i_w0)
    pltpu.sync_copy(word_s1.at[wid], i_w1)
    pltpu.sync_copy(word_s2.at[wid], i_w2)

    ctx_base = wid * (n_ctx * CHUNK)
    neg_base = wid * (n_neg * CHUNK)
    word_base = wid * CHUNK

    def ctx_chunk(i, carry):
        cp = [
            pltpu.async_copy(emb_w.at[i_cs0.at[i]], b_e0, s_e0),
            pltpu.async_copy(emb_w.at[i_cs1.at[i]], b_e1, s_e1),
            pltpu.async_copy(emb_w.at[i_cs2.at[i]], b_e2, s_e2),
            pltpu.async_copy(dis_w.at[i_cs0.at[i]], b_d0, s_d0),
            pltpu.async_copy(dis_w.at[i_cs1.at[i]], b_d1, s_d1),
            pltpu.async_copy(dis_w.at[i_cs2.at[i]], b_d2, s_d2),
            pltpu.async_copy(glob.at[i_cg.at[i]], b_g, s_g),
        ]
        off = ctx_base + i * CHUNK
        for copy, buf, out in zip(
                cp, (b_e0, b_e1, b_e2, b_d0, b_d1, b_d2, b_g),
                (cs0, cs1, cs2, cd0, cd1, cd2, cg)):
            copy.wait()
            pltpu.sync_copy(buf, out.at[pl.ds(off, CHUNK)])
        return carry

    lax.fori_loop(0, n_ctx, ctx_chunk, 0)

    def neg_chunk(i, carry):
        cp_g = pltpu.async_copy(glob.at[i_ng.at[i]], b_g, s_g)
        cp_g.wait()
        pltpu.sync_copy(b_g, ng.at[pl.ds(neg_base + i * CHUNK, CHUNK)])
        return carry

    lax.fori_loop(0, n_neg, neg_chunk, 0)

    cp = [
        pltpu.async_copy(emb_w.at[i_w0.at[0]], b_e0, s_e0),
        pltpu.async_copy(emb_w.at[i_w1.at[0]], b_e1, s_e1),
        pltpu.async_copy(emb_w.at[i_w2.at[0]], b_e2, s_e2),
        pltpu.async_copy(dis_w.at[i_w0.at[0]], b_d0, s_d0),
        pltpu.async_copy(dis_w.at[i_w1.at[0]], b_d1, s_d1),
        pltpu.async_copy(dis_w.at[i_w2.at[0]], b_d2, s_d2),
    ]
    for copy, buf, out in zip(
            cp, (b_e0, b_e1, b_e2, b_d0, b_d1, b_d2),
            (ws0, ws1, ws2, wd0, wd1, wd2)):
        copy.wait()
        pltpu.sync_copy(buf, out.at[pl.ds(word_base, CHUNK)])


def _sc_gather(emb_w, dis_w, glob, ctx_flat, neg_flat, word_ids):
    bc = ctx_flat.shape[0]
    bn = neg_flat.shape[0]
    b = word_ids.shape[0]
    nck = bc // (NW * CHUNK)
    nnk = bn // (NW * CHUNK)

    def part(a, nk):
        return a.reshape(NW, nk, CHUNK)

    ctx3 = ctx_flat * NUM_SENSE
    w3 = word_ids * NUM_SENSE
    idx_in = (part(ctx3, nck), part(ctx3 + 1, nck), part(ctx3 + 2, nck),
              part(ctx_flat, nck), part(neg_flat, nnk),
              part(w3, 1), part(w3 + 1, 1), part(w3 + 2, 1))
    f32 = jnp.float32
    row = jax.ShapeDtypeStruct((bc, EMB_DIM), f32)
    roww = jax.ShapeDtypeStruct((b, EMB_DIM), f32)
    out_type = [row] * 6 + [row, jax.ShapeDtypeStruct((bn, EMB_DIM), f32)] + [roww] * 6
    mesh = plsc.VectorSubcoreMesh(core_axis_name="c", subcore_axis_name="s",
                                  num_cores=NC, num_subcores=NS)
    run = pl.kernel(
        _gather_body,
        out_type=out_type,
        mesh=mesh,
        compiler_params=pltpu.CompilerParams(use_tc_tiling_on_sc=False),
        scratch_types=(
            [pltpu.VMEM((nck, CHUNK), jnp.int32)] * 4
            + [pltpu.VMEM((nnk, CHUNK), jnp.int32)]
            + [pltpu.VMEM((1, CHUNK), jnp.int32)] * 3
            + [pltpu.VMEM((CHUNK, EMB_DIM), f32)] * 7
            + [pltpu.SemaphoreType.DMA] * 7
        ),
    )
    return run(emb_w, dis_w, glob, *idx_in)


def _dense_body(cs0, cs1, cs2, cd0, cd1, cd2, cg, ng,
                ws0, ws1, ws2, wd0, wd1, wd2, out_ref, *, nblocks, denom):
    i = pl.program_id(0)

    @pl.when(i == 0)
    def _():
        out_ref[0, 0] = 0.0

    cgv = cg[...]
    v0 = jnp.mean(cgv, axis=1)

    s0, s1, s2 = cs0[...], cs1[...], cs2[...]
    d0, d1, d2 = cd0[...], cd1[...], cd2[...]

    def disamb_pass(v):
        vb = v[:, None, :]
        t0 = jnp.sum(d0 * vb, axis=-1)
        t1 = jnp.sum(d1 * vb, axis=-1)
        t2 = jnp.sum(d2 * vb, axis=-1)
        m = jnp.maximum(jnp.maximum(t0, t1), t2)
        e0 = jnp.exp(t0 - m)
        e1 = jnp.exp(t1 - m)
        e2 = jnp.exp(t2 - m)
        inv = 1.0 / (e0 + e1 + e2)
        u = (s0 * (e0 * inv)[..., None] + s1 * (e1 * inv)[..., None]
             + s2 * (e2 * inv)[..., None])
        return jnp.mean(u, axis=1)

    v3 = disamb_pass(disamb_pass(v0))

    tw0 = jnp.sum(wd0[...] * v3, axis=-1, keepdims=True)
    tw1 = jnp.sum(wd1[...] * v3, axis=-1, keepdims=True)
    tw2 = jnp.sum(wd2[...] * v3, axis=-1, keepdims=True)
    mw = jnp.maximum(jnp.maximum(tw0, tw1), tw2)
    ew0 = jnp.exp(tw0 - mw)
    ew1 = jnp.exp(tw1 - mw)
    ew2 = jnp.exp(tw2 - mw)
    invw = 1.0 / (ew0 + ew1 + ew2)
    a0 = ew0 * invw
    a1 = ew1 * invw
    a2 = ew2 * invw

    def sigdots(rows):
        p0 = jax.nn.sigmoid(jnp.sum(rows * ws0[...][:, None, :], axis=-1))
        p1 = jax.nn.sigmoid(jnp.sum(rows * ws1[...][:, None, :], axis=-1))
        p2 = jax.nn.sigmoid(jnp.sum(rows * ws2[...][:, None, :], axis=-1))
        return a0 * p0 + a1 * p1 + a2 * p2

    p = sigdots(cgv)
    q = sigdots(ng[...])
    part = (-jnp.sum(jnp.log(jnp.maximum(p, _TINY)))
            - jnp.sum(jnp.log(jnp.maximum(1.0 - q, _TINY))))
    out_ref[0, 0] += part

    @pl.when(i == nblocks - 1)
    def _():
        out_ref[0, 0] = out_ref[0, 0] * denom


def _dense_loss(cs, cd, cg, ng, ws, wd, block_b):
    b, c, d = cg.shape
    n = ng.shape[1]
    nblocks = b // block_b
    big = pl.BlockSpec((block_b, c, d), lambda i: (i, 0, 0))
    negs = pl.BlockSpec((block_b, n, d), lambda i: (i, 0, 0))
    small = pl.BlockSpec((block_b, d), lambda i: (i, 0))
    out = pl.pallas_call(
        functools.partial(_dense_body, nblocks=nblocks, denom=1.0 / (b * c)),
        grid=(nblocks,),
        in_specs=[big] * 6 + [big, negs] + [small] * 6,
        out_specs=pl.BlockSpec((1, 1), lambda i: (0, 0), memory_space=pltpu.SMEM),
        out_shape=jax.ShapeDtypeStruct((1, 1), jnp.float32),
    )(*cs, *cd, cg, ng, *ws, *wd)
    return out[0, 0]


def kernel(word_ids, context_ids, context_masks, neg_ids, emb_w, global_w, disamb_w):
    del context_masks
    b, c = context_ids.shape
    n = neg_ids.shape[1]
    ctx_flat = context_ids.reshape(-1)
    neg_flat = neg_ids.reshape(-1)
    (cs0, cs1, cs2, cd0, cd1, cd2, cg, ng,
     ws0, ws1, ws2, wd0, wd1, wd2) = _sc_gather(
        emb_w, disamb_w, global_w, ctx_flat, neg_flat, word_ids)
    cs = [x.reshape(b, c, EMB_DIM) for x in (cs0, cs1, cs2)]
    cd = [x.reshape(b, c, EMB_DIM) for x in (cd0, cd1, cd2)]
    cg = cg.reshape(b, c, EMB_DIM)
    ng = ng.reshape(b, n, EMB_DIM)
    return _dense_loss(cs, cd, cg, ng,
                       (ws0, ws1, ws2), (wd0, wd1, wd2), block_b=128)


# speedup vs baseline: 2.5390x; 1.3620x over previous
import functools

import jax
import jax.numpy as jnp
import numpy as np
from jax import lax
from jax.experimental import pallas as pl
from jax.experimental.pallas import tpu as pltpu
from jax.experimental.pallas import tpu_sc as plsc

NUM_SENSE = 3
EMB_DIM = 64
LANES = 16
NK = EMB_DIM // LANES

NC = 2
NS = 16
NW = NC * NS
EB = 4

_TINY = float(np.finfo(np.float32).tiny)


def _vdot(ref, row, v4):
    acc = ref[row, pl.ds(0, LANES)] * v4[0]
    for k in range(1, NK):
        acc = acc + ref[row, pl.ds(k * LANES, LANES)] * v4[k]
    return jnp.sum(acc)


def _row4(ref, row):
    return [ref[row, pl.ds(k * LANES, LANES)] for k in range(NK)]


def _store_scalar_vec(ref, idx, splat_vec):
    lane = lax.iota(jnp.int32, LANES)
    plsc.store_scatter(ref, [jnp.full((LANES,), idx, jnp.int32)],
                       splat_vec, mask=lane == 0)


def _softmax3_vec(t0, t1, t2):
    m = jnp.maximum(jnp.maximum(t0, t1), t2)
    e0 = jnp.exp(jnp.broadcast_to(t0 - m, (LANES,)))
    e1 = jnp.exp(jnp.broadcast_to(t1 - m, (LANES,)))
    e2 = jnp.exp(jnp.broadcast_to(t2 - m, (LANES,)))
    inv = 1.0 / (e0 + e1 + e2)
    return e0 * inv, e1 * inv, e2 * inv


def _sc_body(emb_w, dis_w, glob, ictx0, ictx1, ictx2, icg, iw, ing,
             p_out, q_out,
             v_ictx0, v_ictx1, v_ictx2, v_icg, v_iw, v_ing,
             be0, be1, be2, bd0, bd1, bd2, bg, bwe, bwd, bng,
             pbuf, qbuf,
             s_e0, s_e1, s_e2, s_d0, s_d1, s_d2, s_g, s_we, s_wd, s_ng,
             *, c, n, n_eb):
    wid = lax.axis_index("s") * NC + lax.axis_index("c")
    ecb = EB * c
    enb = EB * n
    ewb = EB * NUM_SENSE

    pltpu.sync_copy(ictx0.at[wid], v_ictx0)
    pltpu.sync_copy(ictx1.at[wid], v_ictx1)
    pltpu.sync_copy(ictx2.at[wid], v_ictx2)
    pltpu.sync_copy(icg.at[wid], v_icg)
    pltpu.sync_copy(iw.at[wid], v_iw)
    pltpu.sync_copy(ing.at[wid], v_ing)

    def eb_step(i, carry):
        cps = [
            pltpu.async_copy(dis_w.at[v_ictx0.at[i]], bd0, s_d0),
            pltpu.async_copy(dis_w.at[v_ictx1.at[i]], bd1, s_d1),
            pltpu.async_copy(dis_w.at[v_ictx2.at[i]], bd2, s_d2),
            pltpu.async_copy(glob.at[v_icg.at[i]], bg, s_g),
            pltpu.async_copy(emb_w.at[v_ictx0.at[i]], be0, s_e0),
            pltpu.async_copy(emb_w.at[v_ictx1.at[i]], be1, s_e1),
            pltpu.async_copy(emb_w.at[v_ictx2.at[i]], be2, s_e2),
            pltpu.async_copy(emb_w.at[v_iw.at[i]], bwe, s_we),
            pltpu.async_copy(dis_w.at[v_iw.at[i]], bwd, s_wd),
            pltpu.async_copy(glob.at[v_ing.at[i]], bng, s_ng),
        ]
        for cp in cps:
            cp.wait()

        for b in range(EB):
            base_c = b * c
            def cacc(cc, acc4):
                r4 = _row4(bg, base_c + cc)
                return [acc4[k] + r4[k] for k in range(NK)]
            v0 = lax.fori_loop(0, c, cacc, [jnp.zeros((LANES,), jnp.float32)] * NK)
            v0 = [x * (1.0 / c) for x in v0]

            def dis_pass(v4):
                def body(cc, u4):
                    row = base_c + cc
                    a0, a1, a2 = _softmax3_vec(
                        _vdot(bd0, row, v4), _vdot(bd1, row, v4), _vdot(bd2, row, v4))
                    return [u4[k]
                            + a0 * be0[row, pl.ds(k * LANES, LANES)]
                            + a1 * be1[row, pl.ds(k * LANES, LANES)]
                            + a2 * be2[row, pl.ds(k * LANES, LANES)]
                            for k in range(NK)]
                u4 = lax.fori_loop(0, c, body, [jnp.zeros((LANES,), jnp.float32)] * NK)
                return [x * (1.0 / c) for x in u4]

            v3 = dis_pass(dis_pass(v0))

            wrow = b * NUM_SENSE
            aw0, aw1, aw2 = _softmax3_vec(
                _vdot(bwd, wrow, v3), _vdot(bwd, wrow + 1, v3), _vdot(bwd, wrow + 2, v3))
            we0 = _row4(bwe, wrow)
            we1 = _row4(bwe, wrow + 1)
            we2 = _row4(bwe, wrow + 2)

            def mix_prob(rows_ref, row):
                t0 = _vdot(rows_ref, row, we0)
                t1 = _vdot(rows_ref, row, we1)
                t2 = _vdot(rows_ref, row, we2)
                g0 = 1.0 / (1.0 + jnp.exp(jnp.broadcast_to(-t0, (LANES,))))
                g1 = 1.0 / (1.0 + jnp.exp(jnp.broadcast_to(-t1, (LANES,))))
                g2 = 1.0 / (1.0 + jnp.exp(jnp.broadcast_to(-t2, (LANES,))))
                return aw0 * g0 + aw1 * g1 + aw2 * g2

            def pos_body(cc, carry):
                _store_scalar_vec(pbuf, i * ecb + base_c + cc,
                                  mix_prob(bg, base_c + cc))
                return carry

            lax.fori_loop(0, c, pos_body, 0)

            def neg_body(nn, carry):
                _store_scalar_vec(qbuf, i * enb + b * n + nn,
                                  mix_prob(bng, b * n + nn))
                return carry

            lax.fori_loop(0, n, neg_body, 0)
        return carry

    lax.fori_loop(0, n_eb, eb_step, 0)

    pltpu.sync_copy(pbuf, p_out.at[pl.ds(wid * (n_eb * ecb), n_eb * ecb)])
    pltpu.sync_copy(qbuf, q_out.at[pl.ds(wid * (n_eb * enb), n_eb * enb)])


def _sc_probs(emb_w, dis_w, glob, word_ids, context_ids, neg_ids):
    b, c = context_ids.shape
    n = neg_ids.shape[1]
    wb = b // NW
    n_eb = wb // EB
    ecb, enb, ewb = EB * c, EB * n, EB * NUM_SENSE

    ctx3 = context_ids * NUM_SENSE
    wsen = word_ids[:, None] * NUM_SENSE + jnp.arange(NUM_SENSE, dtype=word_ids.dtype)

    ictx0 = ctx3.reshape(NW, n_eb, ecb)
    ictx1 = (ctx3 + 1).reshape(NW, n_eb, ecb)
    ictx2 = (ctx3 + 2).reshape(NW, n_eb, ecb)
    icg = context_ids.reshape(NW, n_eb, ecb)
    iw = wsen.reshape(NW, n_eb, ewb)
    ing = neg_ids.reshape(NW, n_eb, enb)

    f32 = jnp.float32
    mesh = plsc.VectorSubcoreMesh(core_axis_name="c", subcore_axis_name="s",
                                  num_cores=NC, num_subcores=NS)
    run = pl.kernel(
        functools.partial(_sc_body, c=c, n=n, n_eb=n_eb),
        out_type=[jax.ShapeDtypeStruct((b * c,), f32),
                  jax.ShapeDtypeStruct((b * n,), f32)],
        mesh=mesh,
        compiler_params=pltpu.CompilerParams(use_tc_tiling_on_sc=False,
                                             needs_layout_passes=False),
        scratch_types=(
            [pltpu.VMEM((n_eb, ecb), jnp.int32)] * 4
            + [pltpu.VMEM((n_eb, ewb), jnp.int32),
               pltpu.VMEM((n_eb, enb), jnp.int32)]
            + [pltpu.VMEM((ecb, EMB_DIM), f32)] * 7
            + [pltpu.VMEM((ewb, EMB_DIM), f32)] * 2
            + [pltpu.VMEM((enb, EMB_DIM), f32),
               pltpu.VMEM((wb * c,), f32),
               pltpu.VMEM((wb * n,), f32)]
            + [pltpu.SemaphoreType.DMA] * 10
        ),
    )
    return run(emb_w, dis_w, glob, ictx0, ictx1, ictx2, icg, iw, ing)


def _loss_body(p_ref, q_ref, out_ref, *, denom):
    p = p_ref[...]
    q = q_ref[...]
    loss = (-jnp.sum(jnp.log(jnp.maximum(p, _TINY)))
            - jnp.sum(jnp.log(jnp.maximum(1.0 - q, _TINY))))
    out_ref[0, 0] = loss * denom


def _loss(p, q, b, c, n):
    out = pl.pallas_call(
        functools.partial(_loss_body, denom=1.0 / (b * c)),
        in_specs=[pl.BlockSpec((b, c), lambda: (0, 0)),
                  pl.BlockSpec((b, n), lambda: (0, 0))],
        out_specs=pl.BlockSpec((1, 1), lambda: (0, 0), memory_space=pltpu.SMEM),
        out_shape=jax.ShapeDtypeStruct((1, 1), jnp.float32),
    )(p.reshape(b, c), q.reshape(b, n))
    return out[0, 0]


def kernel(word_ids, context_ids, context_masks, neg_ids, emb_w, global_w, disamb_w):
    del context_masks
    b, c = context_ids.shape
    n = neg_ids.shape[1]
    p, q = _sc_probs(emb_w, disamb_w, global_w, word_ids, context_ids, neg_ids)
    return _loss(p, q, b, c, n)


# speedup vs baseline: 2.7506x; 1.0833x over previous
import functools

import jax
import jax.numpy as jnp
import numpy as np
from jax import lax
from jax.experimental import pallas as pl
from jax.experimental.pallas import tpu as pltpu
from jax.experimental.pallas import tpu_sc as plsc

NUM_SENSE = 3
EMB_DIM = 64
LANES = 16
NK = EMB_DIM // LANES

NC = 2
NS = 16
NW = NC * NS
EB = 4

_TINY = float(np.finfo(np.float32).tiny)


def _pad8(x):
    return (x + 7) // 8 * 8


def _vdot(ref, row, v4):
    acc = ref[row, pl.ds(0, LANES)] * v4[0]
    for k in range(1, NK):
        acc = acc + ref[row, pl.ds(k * LANES, LANES)] * v4[k]
    return jnp.sum(acc)


def _row4(ref, row):
    return [ref[row, pl.ds(k * LANES, LANES)] for k in range(NK)]


def _store_scalar_vec(ref, idx, splat_vec):
    lane = lax.iota(jnp.int32, LANES)
    plsc.store_scatter(ref, [jnp.full((LANES,), idx, jnp.int32)],
                       splat_vec, mask=lane == 0)


def _softmax3_vec(t0, t1, t2):
    m = jnp.maximum(jnp.maximum(t0, t1), t2)
    e0 = jnp.exp(jnp.broadcast_to(t0 - m, (LANES,)))
    e1 = jnp.exp(jnp.broadcast_to(t1 - m, (LANES,)))
    e2 = jnp.exp(jnp.broadcast_to(t2 - m, (LANES,)))
    inv = 1.0 / (e0 + e1 + e2)
    return e0 * inv, e1 * inv, e2 * inv


def _sc_body(emb_w, dis_w, glob, ictx0, ictx1, ictx2, icg, iw, ing,
             p_out, q_out,
             v_ictx0, v_ictx1, v_ictx2, v_icg, v_iw, v_ing,
             bufs_a, bufs_b, pbuf, qbuf, sems_a, sems_b,
             *, c, n, n_eb):
    wid = lax.axis_index("s") * NC + lax.axis_index("c")
    ecb = EB * c
    enb = EB * n
    ewb = EB * NUM_SENSE
    enb_p = _pad8(enb)
    ewb_p = _pad8(ewb)
    wctx = n_eb * ecb
    wneg = n_eb * enb
    wwrd = n_eb * ewb

    pltpu.sync_copy(ictx0.at[pl.ds(wid * wctx, wctx)], v_ictx0)
    pltpu.sync_copy(ictx1.at[pl.ds(wid * wctx, wctx)], v_ictx1)
    pltpu.sync_copy(ictx2.at[pl.ds(wid * wctx, wctx)], v_ictx2)
    pltpu.sync_copy(icg.at[pl.ds(wid * wctx, wctx)], v_icg)
    pltpu.sync_copy(iw.at[pl.ds(wid * (n_eb * ewb_p), n_eb * ewb_p)], v_iw)
    pltpu.sync_copy(ing.at[pl.ds(wid * (n_eb * enb_p), n_eb * enb_p)], v_ing)

    def streams(i, bufs, sems):
        be0, be1, be2, bd0, bd1, bd2, bg, bng, bwe, bwd = bufs
        c0 = v_ictx0.at[pl.ds(i * ecb, ecb)]
        c1 = v_ictx1.at[pl.ds(i * ecb, ecb)]
        c2 = v_ictx2.at[pl.ds(i * ecb, ecb)]
        gg = v_icg.at[pl.ds(i * ecb, ecb)]
        nn = v_ing.at[pl.ds(i * enb_p, enb_p)]
        ww = v_iw.at[pl.ds(i * ewb_p, ewb_p)]
        return ((dis_w.at[c0], bd0, sems[0]), (dis_w.at[c1], bd1, sems[1]),
                (dis_w.at[c2], bd2, sems[2]), (glob.at[gg], bg, sems[3]),
                (emb_w.at[c0], be0, sems[4]), (emb_w.at[c1], be1, sems[5]),
                (emb_w.at[c2], be2, sems[6]), (glob.at[nn], bng, sems[7]),
                (emb_w.at[ww], bwe, sems[8]), (dis_w.at[ww], bwd, sems[9]))

    def issue(i, bufs, sems):
        for src, dst, sem in streams(i, bufs, sems):
            pltpu.async_copy(src, dst, sem)

    def wait(i, bufs, sems):
        for src, dst, sem in streams(i, bufs, sems):
            pltpu.make_async_copy(src, dst, sem).wait()

    def compute(i, bufs):
        be0, be1, be2, bd0, bd1, bd2, bg, bng, bwe, bwd = bufs
        for b in range(EB):
            base_c = b * c

            def cacc(cc, acc4):
                r4 = _row4(bg, base_c + cc)
                return [acc4[k] + r4[k] for k in range(NK)]
            v0 = lax.fori_loop(0, c, cacc, [jnp.zeros((LANES,), jnp.float32)] * NK)
            v0 = [x * (1.0 / c) for x in v0]

            def dis_pass(v4):
                def body(cc, u4):
                    row = base_c + cc
                    a0, a1, a2 = _softmax3_vec(
                        _vdot(bd0, row, v4), _vdot(bd1, row, v4), _vdot(bd2, row, v4))
                    return [u4[k]
                            + a0 * be0[row, pl.ds(k * LANES, LANES)]
                            + a1 * be1[row, pl.ds(k * LANES, LANES)]
                            + a2 * be2[row, pl.ds(k * LANES, LANES)]
                            for k in range(NK)]
                u4 = lax.fori_loop(0, c, body, [jnp.zeros((LANES,), jnp.float32)] * NK)
                return [x * (1.0 / c) for x in u4]

            v3 = dis_pass(dis_pass(v0))

            wrow = b * NUM_SENSE
            aw0, aw1, aw2 = _softmax3_vec(
                _vdot(bwd, wrow, v3), _vdot(bwd, wrow + 1, v3), _vdot(bwd, wrow + 2, v3))
            we0 = _row4(bwe, wrow)
            we1 = _row4(bwe, wrow + 1)
            we2 = _row4(bwe, wrow + 2)

            def mix_prob(rows_ref, row):
                t0 = _vdot(rows_ref, row, we0)
                t1 = _vdot(rows_ref, row, we1)
                t2 = _vdot(rows_ref, row, we2)
                g0 = 1.0 / (1.0 + jnp.exp(jnp.broadcast_to(-t0, (LANES,))))
                g1 = 1.0 / (1.0 + jnp.exp(jnp.broadcast_to(-t1, (LANES,))))
                g2 = 1.0 / (1.0 + jnp.exp(jnp.broadcast_to(-t2, (LANES,))))
                return aw0 * g0 + aw1 * g1 + aw2 * g2

            def pos_body(cc, carry):
                _store_scalar_vec(pbuf, i * ecb + base_c + cc,
                                  mix_prob(bg, base_c + cc))
                return carry

            lax.fori_loop(0, c, pos_body, 0)

            def neg_body(nb, carry):
                _store_scalar_vec(qbuf, i * enb + b * n + nb,
                                  mix_prob(bng, b * n + nb))
                return carry

            lax.fori_loop(0, n, neg_body, 0)

    issue(0, bufs_a, sems_a)

    def pair_step(i2, carry):
        c0 = 2 * i2
        issue(c0 + 1, bufs_b, sems_b)
        wait(c0, bufs_a, sems_a)
        compute(c0, bufs_a)

        @pl.when(c0 + 2 < n_eb)
        def _():
            issue(c0 + 2, bufs_a, sems_a)

        wait(c0 + 1, bufs_b, sems_b)
        compute(c0 + 1, bufs_b)
        return carry

    lax.fori_loop(0, n_eb // 2, pair_step, 0)

    pltpu.sync_copy(pbuf, p_out.at[pl.ds(wid * wctx, wctx)])
    pltpu.sync_copy(qbuf, q_out.at[pl.ds(wid * wneg, wneg)])


def _sc_probs(emb_w, dis_w, glob, word_ids, context_ids, neg_ids):
    b, c = context_ids.shape
    n = neg_ids.shape[1]
    wb = b // NW
    n_eb = wb // EB
    ecb, enb, ewb = EB * c, EB * n, EB * NUM_SENSE

    enb_p = _pad8(enb)
    ewb_p = _pad8(ewb)
    ctx_flat = context_ids.reshape(-1)
    ctx3 = ctx_flat * NUM_SENSE
    wsen = (word_ids[:, None] * NUM_SENSE
            + jnp.arange(NUM_SENSE, dtype=word_ids.dtype)).reshape(b // EB, ewb)
    iw = jnp.concatenate(
        [wsen, jnp.zeros((b // EB, ewb_p - ewb), wsen.dtype)], axis=1).reshape(-1)
    ing = jnp.concatenate(
        [neg_ids.reshape(b // EB, enb),
         jnp.zeros((b // EB, enb_p - enb), neg_ids.dtype)], axis=1).reshape(-1)

    f32 = jnp.float32
    mesh = plsc.VectorSubcoreMesh(core_axis_name="c", subcore_axis_name="s",
                                  num_cores=NC, num_subcores=NS)

    def bufset():
        return ([pltpu.VMEM((ecb, EMB_DIM), f32)] * 7
                + [pltpu.VMEM((enb_p, EMB_DIM), f32)]
                + [pltpu.VMEM((ewb_p, EMB_DIM), f32)] * 2)

    run = pl.kernel(
        functools.partial(_sc_body, c=c, n=n, n_eb=n_eb),
        out_type=[jax.ShapeDtypeStruct((b * c,), f32),
                  jax.ShapeDtypeStruct((b * n,), f32)],
        mesh=mesh,
        compiler_params=pltpu.CompilerParams(use_tc_tiling_on_sc=False,
                                             needs_layout_passes=False),
        scratch_types=(
            [pltpu.VMEM((wb * c,), jnp.int32)] * 4
            + [pltpu.VMEM((n_eb * ewb_p,), jnp.int32),
               pltpu.VMEM((n_eb * enb_p,), jnp.int32)]
            + [bufset(), bufset()]
            + [pltpu.VMEM((wb * c,), f32),
               pltpu.VMEM((wb * n,), f32)]
            + [[pltpu.SemaphoreType.DMA] * 10, [pltpu.SemaphoreType.DMA] * 10]
        ),
    )
    return run(emb_w, dis_w, glob, ctx3, ctx3 + 1, ctx3 + 2, ctx_flat, iw, ing)


def _loss_body(p_ref, q_ref, out_ref, *, denom):
    p = p_ref[...]
    q = q_ref[...]
    loss = (-jnp.sum(jnp.log(jnp.maximum(p, _TINY)))
            - jnp.sum(jnp.log(jnp.maximum(1.0 - q, _TINY))))
    out_ref[0, 0] = loss * denom


def _loss(p, q, b, c, n):
    out = pl.pallas_call(
        functools.partial(_loss_body, denom=1.0 / (b * c)),
        in_specs=[pl.BlockSpec((b, c), lambda: (0, 0)),
                  pl.BlockSpec((b, n), lambda: (0, 0))],
        out_specs=pl.BlockSpec((1, 1), lambda: (0, 0), memory_space=pltpu.SMEM),
        out_shape=jax.ShapeDtypeStruct((1, 1), jnp.float32),
    )(p.reshape(b, c), q.reshape(b, n))
    return out[0, 0]


def kernel(word_ids, context_ids, context_masks, neg_ids, emb_w, global_w, disamb_w):
    del context_masks
    b, c = context_ids.shape
    n = neg_ids.shape[1]
    p, q = _sc_probs(emb_w, disamb_w, global_w, word_ids, context_ids, neg_ids)
    return _loss(p, q, b, c, n)
